# baseline (speedup 1.0000x reference)
import functools

import jax
import jax.numpy as jnp
from jax.experimental import pallas as pl
from jax.experimental.pallas import tpu as pltpu

N, D, G = 100000, 128, 512
B = 2000
NB = N // B

_NEG = -1e30


def _onehot_mask(b, n_rows):
    return b[:, None] == jax.lax.broadcasted_iota(jnp.int32, (n_rows, G), 1)


def _stage_a_kernel(x_ref, b3_ref, wg_ref, bg_ref, gate_ref, segmax_ref, smax_acc):
    i = pl.program_id(0)

    @pl.when(i == 0)
    def _():
        smax_acc[...] = jnp.full((1, G), _NEG, jnp.float32)

    x = x_ref[...]
    gate = jax.lax.dot_general(
        x, wg_ref[...],
        (((1,), (0,)), ((), ())),
        preferred_element_type=jnp.float32) + bg_ref[0, 0]
    gate_ref[...] = gate.reshape(1, 1, B)

    mask = _onehot_mask(b3_ref[0, 0, :], B)
    masked = jnp.where(mask, gate, _NEG)
    smax_acc[...] = jnp.maximum(smax_acc[...],
                                jnp.max(masked, axis=0, keepdims=True))

    @pl.when(i == NB - 1)
    def _():
        segmax_ref[...] = smax_acc[...].reshape(G, 1)


def _stage_b_kernel(gate_ref, b3_ref, segmax_ref, e_ref, recip_ref, den_acc):
    i = pl.program_id(0)

    @pl.when(i == 0)
    def _():
        den_acc[...] = jnp.zeros((G, 1), jnp.float32)

    mask = _onehot_mask(b3_ref[0, 0, :], B)
    maskf = mask.astype(jnp.float32)
    max_row = jax.lax.dot_general(
        maskf, segmax_ref[...],
        (((1,), (0,)), ((), ())),
        preferred_element_type=jnp.float32)
    gate = gate_ref[...].reshape(B, 1)
    e = jnp.exp(gate - max_row)
    e_ref[...] = e.reshape(1, 1, B)
    den_acc[...] += jax.lax.dot_general(
        maskf, e,
        (((0,), (0,)), ((), ())),
        preferred_element_type=jnp.float32)

    @pl.when(i == NB - 1)
    def _():
        recip_ref[...] = 1.0 / (den_acc[...] + 1e-16)


def _stage_c_kernel(x_ref, e3_ref, b3_ref, recip_ref, g_ref, out_ref, out_acc):
    i = pl.program_id(0)

    @pl.when(i == 0)
    def _():
        out_acc[...] = jnp.zeros((G, D), jnp.float32)

    mask = _onehot_mask(b3_ref[0, 0, :], B)
    maskf = mask.astype(jnp.float32)
    r_row = jax.lax.dot_general(
        maskf, recip_ref[...],
        (((1,), (0,)), ((), ())),
        preferred_element_type=jnp.float32)
    g = e3_ref[...].reshape(B, 1) * r_row
    g_ref[...] = g.reshape(1, 1, B)

    vals = x_ref[...] * g
    out_acc[...] += jax.lax.dot_general(
        mask.astype(jnp.bfloat16), vals.astype(jnp.bfloat16),
        (((0,), (0,)), ((), ())),
        preferred_element_type=jnp.float32)

    @pl.when(i == NB - 1)
    def _():
        out_ref[...] = out_acc[...]


def kernel(x, batch, size, Wg, bg):
    del size
    b3 = batch.astype(jnp.int32).reshape(NB, 1, B)
    bg2 = bg.reshape(1, 1)

    gate3, segmax = pl.pallas_call(
        _stage_a_kernel,
        grid=(NB,),
        in_specs=[
            pl.BlockSpec((B, D), lambda i: (i, 0)),
            pl.BlockSpec((1, 1, B), lambda i: (i, 0, 0)),
            pl.BlockSpec((D, 1), lambda i: (0, 0)),
            pl.BlockSpec((1, 1), lambda i: (0, 0)),
        ],
        out_specs=[
            pl.BlockSpec((1, 1, B), lambda i: (i, 0, 0)),
            pl.BlockSpec((G, 1), lambda i: (0, 0)),
        ],
        out_shape=[
            jax.ShapeDtypeStruct((NB, 1, B), jnp.float32),
            jax.ShapeDtypeStruct((G, 1), jnp.float32),
        ],
        scratch_shapes=[pltpu.VMEM((1, G), jnp.float32)],
    )(x, b3, Wg, bg2)

    e3, recip = pl.pallas_call(
        _stage_b_kernel,
        grid=(NB,),
        in_specs=[
            pl.BlockSpec((1, 1, B), lambda i: (i, 0, 0)),
            pl.BlockSpec((1, 1, B), lambda i: (i, 0, 0)),
            pl.BlockSpec((G, 1), lambda i: (0, 0)),
        ],
        out_specs=[
            pl.BlockSpec((1, 1, B), lambda i: (i, 0, 0)),
            pl.BlockSpec((G, 1), lambda i: (0, 0)),
        ],
        out_shape=[
            jax.ShapeDtypeStruct((NB, 1, B), jnp.float32),
            jax.ShapeDtypeStruct((G, 1), jnp.float32),
        ],
        scratch_shapes=[pltpu.VMEM((G, 1), jnp.float32)],
    )(gate3, b3, segmax)

    g3, out = pl.pallas_call(
        _stage_c_kernel,
        grid=(NB,),
        in_specs=[
            pl.BlockSpec((B, D), lambda i: (i, 0)),
            pl.BlockSpec((1, 1, B), lambda i: (i, 0, 0)),
            pl.BlockSpec((1, 1, B), lambda i: (i, 0, 0)),
            pl.BlockSpec((G, 1), lambda i: (0, 0)),
        ],
        out_specs=[
            pl.BlockSpec((1, 1, B), lambda i: (i, 0, 0)),
            pl.BlockSpec((G, D), lambda i: (0, 0)),
        ],
        out_shape=[
            jax.ShapeDtypeStruct((NB, 1, B), jnp.float32),
            jax.ShapeDtypeStruct((G, D), jnp.float32),
        ],
        scratch_shapes=[pltpu.VMEM((G, D), jnp.float32)],
    )(x, e3, b3, recip)

    g = g3.reshape(N, 1)
    return (out, g)


# speedup vs baseline: 1.8211x; 1.8211x over previous
import functools

import jax
import jax.numpy as jnp
from jax import lax
from jax.experimental import pallas as pl
from jax.experimental.pallas import tpu as pltpu
from jax.experimental.pallas import tpu_sc as plsc

N, D, G = 100000, 128, 512
B = 2000
NB = N // B

NW = 32
RT = 3136
NPAD = NW * RT
NCH = RT // 16

_NEG = -1e30


def _onehot_mask(b, n_rows):
    return b[:, None] == jax.lax.broadcasted_iota(jnp.int32, (n_rows, G), 1)


def _stage_a_kernel(x_ref, b3_ref, wg_ref, bg_ref, gate_ref, segmax_ref, smax_acc):
    i = pl.program_id(0)

    @pl.when(i == 0)
    def _():
        smax_acc[...] = jnp.full((1, G), _NEG, jnp.float32)

    x = x_ref[...]
    w = wg_ref[...][:, 0]
    gate = jnp.sum(x * w[None, :], axis=1, keepdims=True) + bg_ref[0, 0]
    gate_ref[...] = gate.reshape(1, 1, B)

    mask = _onehot_mask(b3_ref[0, 0, :], B)
    masked = jnp.where(mask, gate, _NEG)
    smax_acc[...] = jnp.maximum(smax_acc[...],
                                jnp.max(masked, axis=0, keepdims=True))

    @pl.when(i == NB - 1)
    def _():
        segmax_ref[...] = smax_acc[...].reshape(G, 1)


def _sc_stats_body(gate_hbm, batch_hbm, smax_hbm, e_hbm, parts_hbm,
                   gate_loc, batch_loc, e_loc, c_loc, smax_loc,
                   st_loc, en_loc, parts_loc):
    w = lax.axis_index("c") * 16 + lax.axis_index("s")
    base = w * RT
    pltpu.sync_copy(gate_hbm.at[pl.ds(base, RT)], gate_loc)
    pltpu.sync_copy(batch_hbm.at[pl.ds(base, RT)], batch_loc)
    pltpu.sync_copy(smax_hbm, smax_loc)

    iota = lax.iota(jnp.int32, 16)
    zi = jnp.zeros((16,), jnp.int32)
    zf = jnp.zeros((16,), jnp.float32)

    def init_chunk(k, _):
        st_loc[pl.ds(k * 16, 16)] = zi
        en_loc[pl.ds(k * 16, 16)] = zi
        parts_loc[pl.ds(k * 16, 16)] = zf
        return 0
    lax.fori_loop(0, G // 16, init_chunk, 0)

    def bdry_chunk(j, _):
        off = j * 16
        b = batch_loc[pl.ds(off, 16)]
        bp = plsc.load_gather(batch_loc, [jnp.maximum(off + iota - 1, 0)])
        is_b = b != bp
        pos = jnp.full((16,), base + off, jnp.int32) + iota
        plsc.store_scatter(st_loc, [b], pos, mask=is_b)
        plsc.store_scatter(en_loc, [bp], pos, mask=is_b)
        return 0
    lax.fori_loop(0, NCH, bdry_chunk, 0)

    lane0 = iota == 0
    b0 = batch_loc[pl.ds(0, 16)][0]
    bl = batch_loc[pl.ds(RT - 16, 16)][15]
    plsc.store_scatter(st_loc, [jnp.full((16,), b0, jnp.int32)],
                       jnp.full((16,), base, jnp.int32), mask=lane0)
    plsc.store_scatter(en_loc, [jnp.full((16,), bl, jnp.int32)],
                       jnp.full((16,), base + RT, jnp.int32), mask=lane0)

    def e_chunk(j, carry):
        off = j * 16
        g = gate_loc[pl.ds(off, 16)]
        b = batch_loc[pl.ds(off, 16)]
        mx = plsc.load_gather(smax_loc, [b])
        e = jnp.exp(g - mx)
        e_loc[pl.ds(off, 16)] = e
        c_loc[pl.ds(off, 16)] = plsc.cumsum(e) + carry
        return carry + jnp.sum(e)
    lax.fori_loop(0, NCH, e_chunk, jnp.float32(0.0))

    s_lo = b0
    s_hi = bl
    nch = (s_hi - s_lo + 16) // 16

    def part_chunk(k, _):
        s = jnp.full((16,), s_lo + k * 16, jnp.int32) + iota
        m = s <= s_hi
        sc = jnp.minimum(s, G - 1)
        st = plsc.load_gather(st_loc, [sc])
        en = plsc.load_gather(en_loc, [sc])
        lo_l = jnp.clip(st, base, base + RT) - base
        hi_l = jnp.clip(en, base, base + RT) - base
        vh = jnp.where(hi_l > 0,
                       plsc.load_gather(c_loc, [jnp.maximum(hi_l - 1, 0)]), 0.0)
        vl = jnp.where(lo_l > 0,
                       plsc.load_gather(c_loc, [jnp.maximum(lo_l - 1, 0)]), 0.0)
        plsc.store_scatter(parts_loc, [sc], jnp.where(m, vh - vl, 0.0), mask=m)
        return 0
    lax.fori_loop(0, nch, part_chunk, 0)

    pltpu.sync_copy(e_loc, e_hbm.at[pl.ds(base, RT)])
    pltpu.sync_copy(parts_loc, parts_hbm.at[w])


def _sc_g_body(batch_hbm, e_hbm, parts_hbm, g_hbm,
               batch_loc, e_loc, g_loc, parts32_loc, rd_loc):
    w = lax.axis_index("c") * 16 + lax.axis_index("s")
    base = w * RT
    pltpu.sync_copy(batch_hbm.at[pl.ds(base, RT)], batch_loc)
    pltpu.sync_copy(e_hbm.at[pl.ds(base, RT)], e_loc)
    pltpu.sync_copy(parts_hbm, parts32_loc)

    def den_chunk(k, _):
        acc = jnp.zeros((16,), jnp.float32)
        for r in range(NW):
            acc = acc + parts32_loc[r, pl.ds(k * 16, 16)]
        rd_loc[pl.ds(k * 16, 16)] = 1.0 / (acc + 1e-16)
        return 0
    lax.fori_loop(0, G // 16, den_chunk, 0)

    def g_chunk(j, _):
        off = j * 16
        b = batch_loc[pl.ds(off, 16)]
        e = e_loc[pl.ds(off, 16)]
        g_loc[pl.ds(off, 16)] = e * plsc.load_gather(rd_loc, [b])
        return 0
    lax.fori_loop(0, NCH, g_chunk, 0)

    pltpu.sync_copy(g_loc, g_hbm.at[pl.ds(base, RT)])


def _stage_c_kernel(x_ref, g3_ref, b3_ref, out_ref, out_acc):
    i = pl.program_id(0)

    @pl.when(i == 0)
    def _():
        out_acc[...] = jnp.zeros((G, D), jnp.float32)

    mask = _onehot_mask(b3_ref[0, 0, :], B)
    g = g3_ref[...].reshape(B, 1)
    vals = x_ref[...] * g
    out_acc[...] += jax.lax.dot_general(
        mask.astype(jnp.bfloat16), vals.astype(jnp.bfloat16),
        (((0,), (0,)), ((), ())),
        preferred_element_type=jnp.float32)

    @pl.when(i == NB - 1)
    def _():
        out_ref[...] = out_acc[...]


_SC_MESH = plsc.VectorSubcoreMesh(core_axis_name="c", subcore_axis_name="s")

_sc_stats = pl.kernel(
    _sc_stats_body,
    out_type=[
        jax.ShapeDtypeStruct((NPAD,), jnp.float32),
        jax.ShapeDtypeStruct((NW, G), jnp.float32),
    ],
    mesh=_SC_MESH,
    scratch_types=[
        pltpu.VMEM((RT,), jnp.float32),
        pltpu.VMEM((RT,), jnp.int32),
        pltpu.VMEM((RT,), jnp.float32),
        pltpu.VMEM((RT,), jnp.float32),
        pltpu.VMEM((G,), jnp.float32),
        pltpu.VMEM((G,), jnp.int32),
        pltpu.VMEM((G,), jnp.int32),
        pltpu.VMEM((G,), jnp.float32),
    ],
    compiler_params=pltpu.CompilerParams(needs_layout_passes=False),
)

_sc_g = pl.kernel(
    _sc_g_body,
    out_type=jax.ShapeDtypeStruct((NPAD,), jnp.float32),
    mesh=_SC_MESH,
    scratch_types=[
        pltpu.VMEM((RT,), jnp.int32),
        pltpu.VMEM((RT,), jnp.float32),
        pltpu.VMEM((RT,), jnp.float32),
        pltpu.VMEM((NW, G), jnp.float32),
        pltpu.VMEM((G,), jnp.float32),
    ],
    compiler_params=pltpu.CompilerParams(needs_layout_passes=False),
)


def kernel(x, batch, size, Wg, bg):
    del size
    bi = batch.astype(jnp.int32)
    b3 = bi.reshape(NB, 1, B)
    bg2 = bg.reshape(1, 1)

    gate3, segmax = pl.pallas_call(
        _stage_a_kernel,
        grid=(NB,),
        in_specs=[
            pl.BlockSpec((B, D), lambda i: (i, 0)),
            pl.BlockSpec((1, 1, B), lambda i: (i, 0, 0)),
            pl.BlockSpec((D, 1), lambda i: (0, 0)),
            pl.BlockSpec((1, 1), lambda i: (0, 0)),
        ],
        out_specs=[
            pl.BlockSpec((1, 1, B), lambda i: (i, 0, 0)),
            pl.BlockSpec((G, 1), lambda i: (0, 0)),
        ],
        out_shape=[
            jax.ShapeDtypeStruct((NB, 1, B), jnp.float32),
            jax.ShapeDtypeStruct((G, 1), jnp.float32),
        ],
        scratch_shapes=[pltpu.VMEM((1, G), jnp.float32)],
    )(x, b3, Wg, bg2)

    gate_p = jnp.concatenate(
        [gate3.reshape(N), jnp.full((NPAD - N,), _NEG, jnp.float32)])
    batch_p = jnp.concatenate(
        [bi, jnp.full((NPAD - N,), G - 1, jnp.int32)])

    e_p, parts = _sc_stats(gate_p, batch_p, segmax.reshape(G))
    g_p = _sc_g(batch_p, e_p, parts)

    g3 = g_p[:N].reshape(NB, 1, B)
    out = pl.pallas_call(
        _stage_c_kernel,
        grid=(NB,),
        in_specs=[
            pl.BlockSpec((B, D), lambda i: (i, 0)),
            pl.BlockSpec((1, 1, B), lambda i: (i, 0, 0)),
            pl.BlockSpec((1, 1, B), lambda i: (i, 0, 0)),
        ],
        out_specs=pl.BlockSpec((G, D), lambda i: (0, 0)),
        out_shape=jax.ShapeDtypeStruct((G, D), jnp.float32),
        scratch_shapes=[pltpu.VMEM((G, D), jnp.float32)],
    )(x, g3, b3)

    g = g_p[:N].reshape(N, 1)
    return (out, g)
